# baseline (speedup 1.0000x reference)
import functools

import jax
import jax.numpy as jnp
from jax import lax
from jax.experimental import pallas as pl
from jax.experimental.pallas import tpu as pltpu
from jax.experimental.pallas import tpu_sc as plsc

_LANES = 16
_STREAM_ROWS = 2
_CHUNK_STREAMS = 4
_CHUNK_ROWS = _STREAM_ROWS * _CHUNK_STREAMS


def _build_transpose(V, D, NC, NS):
    NW = NC * NS
    UB = 128
    nfull = V // UB
    rem = V - nfull * UB
    per_w = (nfull + NW - 1) // NW

    mesh = plsc.VectorSubcoreMesh(core_axis_name="c", subcore_axis_name="s")

    @functools.partial(
        pl.kernel,
        mesh=mesh,
        compiler_params=pltpu.CompilerParams(
            needs_layout_passes=False, disable_bounds_checks=True),
        out_type=jax.ShapeDtypeStruct((V * D,), jnp.float32),
        scratch_types=[
            pltpu.VMEM((D, UB), jnp.float32),
            pltpu.VMEM((D, UB), jnp.float32),
            pltpu.VMEM((UB * D,), jnp.float32),
            pltpu.VMEM((UB * D,), jnp.float32),
            pltpu.SemaphoreType.DMA,
            pltpu.SemaphoreType.DMA,
            pltpu.SemaphoreType.DMA,
            pltpu.SemaphoreType.DMA,
        ],
    )
    def k1(cvt_hbm, cv_tail, cv_out,
           stg_a, stg_b, tbuf_a, tbuf_b, sem_a, sem_b, sem_oa, sem_ob):
        wid = lax.axis_index("s") * NC + lax.axis_index("c")
        iota = jnp.arange(_LANES, dtype=jnp.int32)
        _ROWCHUNK = 8

        def one_table(src_hbm, tail_hbm, dst_hbm):
            def fire(blk, stg, sem):
                pltpu.async_copy(
                    src_hbm.at[:, pl.ds(blk * UB, UB)], stg, sem)

            def drain(blk, stg, sem):
                pltpu.make_async_copy(
                    src_hbm.at[:, pl.ds(blk * UB, UB)], stg, sem).wait()

            def transpose(stg, tbuf):
                rots = [(iota + d) & (_LANES - 1) for d in range(_LANES)]

                def rows_body(rg, carry):
                    rbase = rg * (4 * _LANES)
                    for half in range(4):
                        for kk in range(D // _LANES):
                            fvec = iota + kk * _LANES
                            for d in range(_LANES):
                                rvec = rots[d] + (rbase + half * _LANES)
                                vals = plsc.load_gather(stg, [fvec, rvec])
                                sidx = rvec * D + fvec
                                plsc.store_scatter(tbuf, [sidx], vals)
                    return carry
                lax.fori_loop(0, UB // (4 * _LANES), rows_body, 0)

            @pl.when(wid < nfull)
            def _():
                fire(wid, stg_a, sem_a)

            def body(j, even, carry):
                blk = wid + j * NW
                nxt = blk + NW

                @pl.when(nxt < nfull)
                def _():
                    fire(nxt, stg_b if even else stg_a,
                         sem_b if even else sem_a)

                @pl.when(blk < nfull)
                def _():
                    cur_stg = stg_a if even else stg_b
                    cur_sem = sem_a if even else sem_b
                    cur_tbuf = tbuf_a if even else tbuf_b
                    cur_osem = sem_oa if even else sem_ob
                    drain(blk, cur_stg, cur_sem)

                    @pl.when(j >= 2)
                    def _():
                        pltpu.make_async_copy(
                            cur_tbuf,
                            dst_hbm.at[pl.ds(blk * UB * D, UB * D)],
                            cur_osem).wait()

                    transpose(cur_stg, cur_tbuf)
                    pltpu.async_copy(
                        cur_tbuf, dst_hbm.at[pl.ds(blk * UB * D, UB * D)],
                        cur_osem)
                return carry

            def body2(jj, carry):
                body(2 * jj, True, carry)
                body(2 * jj + 1, False, carry)
                return carry

            lax.fori_loop(0, (per_w + 1) // 2, body2, 0)

            @pl.when(wid < nfull)
            def _():
                pltpu.make_async_copy(
                    tbuf_a, dst_hbm.at[pl.ds(wid * UB * D, UB * D)],
                    sem_oa).wait()

            @pl.when(wid + NW < nfull)
            def _():
                pltpu.make_async_copy(
                    tbuf_b, dst_hbm.at[pl.ds(wid * UB * D, UB * D)],
                    sem_ob).wait()

            if rem:
                @pl.when(wid == nfull % NW)
                def _():
                    pltpu.sync_copy(tail_hbm, tbuf_a.at[pl.ds(0, rem * D)])
                    pltpu.sync_copy(
                        tbuf_a.at[pl.ds(0, rem * D)],
                        dst_hbm.at[pl.ds(nfull * UB * D, rem * D)])

        one_table(cvt_hbm, cv_tail, cv_out)

    return k1


def _build_main(B, H, D, V, NC, NS):
    NW = NC * NS
    BW = B // NW
    NCH = BW // _CHUNK_ROWS
    SW = _STREAM_ROWS * H
    CE = _CHUNK_ROWS * H

    mesh = plsc.VectorSubcoreMesh(core_axis_name="c", subcore_axis_name="s")

    group_offs = list(range(0, H - _LANES + 1, _LANES))
    if group_offs[-1] != H - _LANES:
        group_offs.append(H - _LANES)

    @functools.partial(
        pl.kernel,
        mesh=mesh,
        compiler_params=pltpu.CompilerParams(
            needs_layout_passes=False, use_tc_tiling_on_sc=False,
            disable_bounds_checks=True),
        out_type=jax.ShapeDtypeStruct((B, H), jnp.float32),
        scratch_types=[
            pltpu.VMEM((BW,), jnp.int32),
            pltpu.VMEM((BW // _STREAM_ROWS, SW), jnp.int32),
            pltpu.VMEM((BW, D), jnp.float32),
            pltpu.VMEM((CE, D), jnp.float32),
            pltpu.VMEM((CE, D), jnp.float32),
            pltpu.VMEM((BW, H), jnp.float32),
            pltpu.SemaphoreType.DMA,
            pltpu.SemaphoreType.DMA,
            pltpu.SemaphoreType.DMA,
        ],
    )
    def k2(uidx_hbm, purch_hbm, cv_hbm, u_hbm, out_hbm,
           uidx_v, purch_v, urows_v, buf_a, buf_b, out_v,
           sem_a, sem_b, sem_u):
        wid = lax.axis_index("s") * NC + lax.axis_index("c")
        base = wid * BW
        iota = jnp.arange(_LANES, dtype=jnp.int32)
        nstr = BW // _STREAM_ROWS
        pltpu.sync_copy(uidx_hbm.at[pl.ds(base, BW)], uidx_v)
        pltpu.sync_copy(purch_hbm.at[pl.ds(wid * nstr, nstr)], purch_v)

        def fire(j, buf, sem):
            for s in range(_CHUNK_STREAMS):
                pltpu.async_copy(
                    cv_hbm.at[purch_v.at[j * _CHUNK_STREAMS + s]],
                    buf.at[pl.ds(s * SW, SW)], sem)

        def drain(j, buf, sem):
            for s in range(_CHUNK_STREAMS):
                pltpu.make_async_copy(
                    cv_hbm.at[purch_v.at[j * _CHUNK_STREAMS + s]],
                    buf.at[pl.ds(s * SW, SW)], sem).wait()

        def compute(j, buf):
            def sub(s, carry):
                for r in range(_STREAM_ROWS):
                    row = j * _CHUNK_ROWS + s * _STREAM_ROWS + r
                    ce0 = s * SW + r * H
                    uvecs = [urows_v[row, pl.ds(kk * _LANES, _LANES)]
                             for kk in range(D // _LANES)]
                    for l0 in group_offs:
                        outvec = jnp.zeros((_LANES,), jnp.float32)
                        for i in range(_LANES):
                            e = ce0 + l0 + i
                            acc = buf[e, pl.ds(0, _LANES)] * uvecs[0]
                            for kk in range(1, D // _LANES):
                                acc = acc + buf[e, pl.ds(kk * _LANES, _LANES)] * uvecs[kk]
                            outvec = jnp.where(iota == i, jnp.sum(acc), outvec)
                        out_v[row, pl.ds(l0, _LANES)] = outvec
                return carry
            lax.fori_loop(0, _CHUNK_STREAMS, sub, 0)

        fire(0, buf_a, sem_a)
        pltpu.async_copy(u_hbm.at[uidx_v], urows_v, sem_u)
        pltpu.make_async_copy(u_hbm.at[uidx_v], urows_v, sem_u).wait()

        def body(jj, carry):
            j0 = 2 * jj
            j1 = 2 * jj + 1
            fire(j1, buf_b, sem_b)
            drain(j0, buf_a, sem_a)
            compute(j0, buf_a)

            @pl.when(j1 + 1 < NCH)
            def _():
                fire(j1 + 1, buf_a, sem_a)

            drain(j1, buf_b, sem_b)
            compute(j1, buf_b)
            return carry

        lax.fori_loop(0, NCH // 2, body, 0)
        pltpu.sync_copy(out_v, out_hbm.at[pl.ds(base, BW)])

    return k2


def kernel(user_idx, user_purchase, character_vector, u):
    B, H = user_purchase.shape
    V, D = character_vector.shape
    info = plsc.get_sparse_core_info()
    NC, NS = info.num_cores, info.num_subcores

    nfull = V // 128
    cv_tail = character_vector[nfull * 128:].reshape(-1)
    k1 = _build_transpose(V, D, NC, NS)
    cv_lin = k1(character_vector.T, cv_tail).reshape(V, D)

    purch2 = user_purchase.reshape(B // _STREAM_ROWS, _STREAM_ROWS * H)
    k2 = _build_main(B, H, D, V, NC, NS)
    return k2(user_idx, purch2, cv_lin, u)


# speedup vs baseline: 1.1099x; 1.1099x over previous
import functools

import jax
import jax.numpy as jnp
from jax import lax
from jax.experimental import pallas as pl
from jax.experimental.pallas import tpu as pltpu
from jax.experimental.pallas import tpu_sc as plsc

_LANES = 16
_STREAM_ROWS = 2
_CHUNK_STREAMS = 4
_CHUNK_ROWS = _STREAM_ROWS * _CHUNK_STREAMS


def _build_transpose(V, D, NC, NS):
    NW = NC * NS
    UB = 128
    nfull = V // UB
    rem = V - nfull * UB
    per_w = (nfull + NW - 1) // NW

    mesh = plsc.VectorSubcoreMesh(core_axis_name="c", subcore_axis_name="s")

    @functools.partial(
        pl.kernel,
        mesh=mesh,
        compiler_params=pltpu.CompilerParams(
            needs_layout_passes=False, disable_bounds_checks=True),
        out_type=jax.ShapeDtypeStruct((V * D,), jnp.float32),
        scratch_types=[
            pltpu.VMEM((D, UB), jnp.float32),
            pltpu.VMEM((D, UB), jnp.float32),
            pltpu.VMEM((UB * D,), jnp.float32),
            pltpu.VMEM((UB * D,), jnp.float32),
            pltpu.SemaphoreType.DMA,
            pltpu.SemaphoreType.DMA,
            pltpu.SemaphoreType.DMA,
            pltpu.SemaphoreType.DMA,
        ],
    )
    def k1(cvt_hbm, cv_tail, cv_out,
           stg_a, stg_b, tbuf_a, tbuf_b, sem_a, sem_b, sem_oa, sem_ob):
        wid = lax.axis_index("s") * NC + lax.axis_index("c")
        iota = jnp.arange(_LANES, dtype=jnp.int32)
        _ROWCHUNK = 8

        def one_table(src_hbm, tail_hbm, dst_hbm):
            def fire(blk, stg, sem):
                pltpu.async_copy(
                    src_hbm.at[:, pl.ds(blk * UB, UB)], stg, sem)

            def drain(blk, stg, sem):
                pltpu.make_async_copy(
                    src_hbm.at[:, pl.ds(blk * UB, UB)], stg, sem).wait()

            def transpose(stg, tbuf):
                rots = [(iota + d) & (_LANES - 1) for d in range(_LANES)]

                def rows_body(rg, carry):
                    rbase = rg * (2 * _LANES)
                    for half in range(2):
                        for kk in range(D // _LANES):
                            fvec = iota + kk * _LANES
                            for d in range(_LANES):
                                rvec = rots[d] + (rbase + half * _LANES)
                                vals = plsc.load_gather(stg, [fvec, rvec])
                                sidx = rvec * D + fvec
                                plsc.store_scatter(tbuf, [sidx], vals)
                    return carry
                lax.fori_loop(0, UB // (2 * _LANES), rows_body, 0)

            @pl.when(wid < nfull)
            def _():
                fire(wid, stg_a, sem_a)

            def body(j, even, carry):
                blk = wid + j * NW
                nxt = blk + NW

                @pl.when(nxt < nfull)
                def _():
                    fire(nxt, stg_b if even else stg_a,
                         sem_b if even else sem_a)

                @pl.when(blk < nfull)
                def _():
                    cur_stg = stg_a if even else stg_b
                    cur_sem = sem_a if even else sem_b
                    cur_tbuf = tbuf_a if even else tbuf_b
                    cur_osem = sem_oa if even else sem_ob
                    drain(blk, cur_stg, cur_sem)

                    @pl.when(j >= 2)
                    def _():
                        pltpu.make_async_copy(
                            cur_tbuf,
                            dst_hbm.at[pl.ds(blk * UB * D, UB * D)],
                            cur_osem).wait()

                    transpose(cur_stg, cur_tbuf)
                    pltpu.async_copy(
                        cur_tbuf, dst_hbm.at[pl.ds(blk * UB * D, UB * D)],
                        cur_osem)
                return carry

            def body2(jj, carry):
                body(2 * jj, True, carry)
                body(2 * jj + 1, False, carry)
                return carry

            lax.fori_loop(0, (per_w + 1) // 2, body2, 0)

            @pl.when(wid < nfull)
            def _():
                pltpu.make_async_copy(
                    tbuf_a, dst_hbm.at[pl.ds(wid * UB * D, UB * D)],
                    sem_oa).wait()

            @pl.when(wid + NW < nfull)
            def _():
                pltpu.make_async_copy(
                    tbuf_b, dst_hbm.at[pl.ds(wid * UB * D, UB * D)],
                    sem_ob).wait()

            if rem:
                @pl.when(wid == nfull % NW)
                def _():
                    pltpu.sync_copy(tail_hbm, tbuf_a.at[pl.ds(0, rem * D)])
                    pltpu.sync_copy(
                        tbuf_a.at[pl.ds(0, rem * D)],
                        dst_hbm.at[pl.ds(nfull * UB * D, rem * D)])

        one_table(cvt_hbm, cv_tail, cv_out)

    return k1


def _build_main(B, H, D, V, NC, NS):
    NW = NC * NS
    BW = B // NW
    NCH = BW // _CHUNK_ROWS
    SW = _STREAM_ROWS * H
    CE = _CHUNK_ROWS * H

    mesh = plsc.VectorSubcoreMesh(core_axis_name="c", subcore_axis_name="s")

    group_offs = list(range(0, H - _LANES + 1, _LANES))
    if group_offs[-1] != H - _LANES:
        group_offs.append(H - _LANES)

    @functools.partial(
        pl.kernel,
        mesh=mesh,
        compiler_params=pltpu.CompilerParams(
            needs_layout_passes=False, use_tc_tiling_on_sc=False,
            disable_bounds_checks=True),
        out_type=jax.ShapeDtypeStruct((B, H), jnp.float32),
        scratch_types=[
            pltpu.VMEM((BW,), jnp.int32),
            pltpu.VMEM((BW // _STREAM_ROWS, SW), jnp.int32),
            pltpu.VMEM((BW, D), jnp.float32),
            pltpu.VMEM((CE, D), jnp.float32),
            pltpu.VMEM((CE, D), jnp.float32),
            pltpu.VMEM((BW, H), jnp.float32),
            pltpu.SemaphoreType.DMA,
            pltpu.SemaphoreType.DMA,
            pltpu.SemaphoreType.DMA,
        ],
    )
    def k2(uidx_hbm, purch_hbm, cv_hbm, u_hbm, out_hbm,
           uidx_v, purch_v, urows_v, buf_a, buf_b, out_v,
           sem_a, sem_b, sem_u):
        wid = lax.axis_index("s") * NC + lax.axis_index("c")
        base = wid * BW
        iota = jnp.arange(_LANES, dtype=jnp.int32)
        nstr = BW // _STREAM_ROWS
        pltpu.sync_copy(uidx_hbm.at[pl.ds(base, BW)], uidx_v)
        pltpu.sync_copy(purch_hbm.at[pl.ds(wid * nstr, nstr)], purch_v)

        def fire(j, buf, sem):
            for s in range(_CHUNK_STREAMS):
                pltpu.async_copy(
                    cv_hbm.at[purch_v.at[j * _CHUNK_STREAMS + s]],
                    buf.at[pl.ds(s * SW, SW)], sem)

        def drain(j, buf, sem):
            for s in range(_CHUNK_STREAMS):
                pltpu.make_async_copy(
                    cv_hbm.at[purch_v.at[j * _CHUNK_STREAMS + s]],
                    buf.at[pl.ds(s * SW, SW)], sem).wait()

        def compute(j, buf):
            def sub(s, carry):
                for r in range(_STREAM_ROWS):
                    row = j * _CHUNK_ROWS + s * _STREAM_ROWS + r
                    ce0 = s * SW + r * H
                    uvecs = [urows_v[row, pl.ds(kk * _LANES, _LANES)]
                             for kk in range(D // _LANES)]
                    for l0 in group_offs:
                        outvec = jnp.zeros((_LANES,), jnp.float32)
                        for i in range(_LANES):
                            e = ce0 + l0 + i
                            acc = buf[e, pl.ds(0, _LANES)] * uvecs[0]
                            for kk in range(1, D // _LANES):
                                acc = acc + buf[e, pl.ds(kk * _LANES, _LANES)] * uvecs[kk]
                            outvec = jnp.where(iota == i, jnp.sum(acc), outvec)
                        out_v[row, pl.ds(l0, _LANES)] = outvec
                return carry
            lax.fori_loop(0, _CHUNK_STREAMS, sub, 0)

        fire(0, buf_a, sem_a)
        pltpu.async_copy(u_hbm.at[uidx_v], urows_v, sem_u)
        pltpu.make_async_copy(u_hbm.at[uidx_v], urows_v, sem_u).wait()

        def body(jj, carry):
            j0 = 2 * jj
            j1 = 2 * jj + 1
            fire(j1, buf_b, sem_b)
            drain(j0, buf_a, sem_a)
            compute(j0, buf_a)

            @pl.when(j1 + 1 < NCH)
            def _():
                fire(j1 + 1, buf_a, sem_a)

            drain(j1, buf_b, sem_b)
            compute(j1, buf_b)
            return carry

        lax.fori_loop(0, NCH // 2, body, 0)
        pltpu.sync_copy(out_v, out_hbm.at[pl.ds(base, BW)])

    return k2


def kernel(user_idx, user_purchase, character_vector, u):
    B, H = user_purchase.shape
    V, D = character_vector.shape
    info = plsc.get_sparse_core_info()
    NC, NS = info.num_cores, info.num_subcores

    nfull = V // 128
    cv_tail = character_vector[nfull * 128:].reshape(-1)
    k1 = _build_transpose(V, D, NC, NS)
    cv_lin = k1(character_vector.T, cv_tail).reshape(V, D)

    purch2 = user_purchase.reshape(B // _STREAM_ROWS, _STREAM_ROWS * H)
    k2 = _build_main(B, H, D, V, NC, NS)
    return k2(user_idx, purch2, cv_lin, u)


# speedup vs baseline: 1.1203x; 1.0094x over previous
import functools

import jax
import jax.numpy as jnp
from jax import lax
from jax.experimental import pallas as pl
from jax.experimental.pallas import tpu as pltpu
from jax.experimental.pallas import tpu_sc as plsc

_LANES = 16
_STREAM_ROWS = 2
_CHUNK_STREAMS = 4
_CHUNK_ROWS = _STREAM_ROWS * _CHUNK_STREAMS


def _build_transpose(V, D, NC, NS):
    NW = NC * NS
    UB = 128
    nfull = V // UB
    rem = V - nfull * UB
    per_w = (nfull + NW - 1) // NW

    mesh = plsc.VectorSubcoreMesh(core_axis_name="c", subcore_axis_name="s")

    @functools.partial(
        pl.kernel,
        mesh=mesh,
        compiler_params=pltpu.CompilerParams(
            needs_layout_passes=False, disable_bounds_checks=True),
        out_type=jax.ShapeDtypeStruct((V * D,), jnp.float32),
        scratch_types=[
            pltpu.VMEM((D, UB), jnp.float32),
            pltpu.VMEM((D, UB), jnp.float32),
            pltpu.VMEM((UB * D,), jnp.float32),
            pltpu.VMEM((UB * D,), jnp.float32),
            pltpu.SemaphoreType.DMA,
            pltpu.SemaphoreType.DMA,
            pltpu.SemaphoreType.DMA,
            pltpu.SemaphoreType.DMA,
        ],
    )
    def k1(cvt_hbm, cv_tail, cv_out,
           stg_a, stg_b, tbuf_a, tbuf_b, sem_a, sem_b, sem_oa, sem_ob):
        wid = lax.axis_index("s") * NC + lax.axis_index("c")
        iota = jnp.arange(_LANES, dtype=jnp.int32)

        def one_table(src_hbm, tail_hbm, dst_hbm):
            def fire(blk, stg, sem):
                pltpu.async_copy(
                    src_hbm.at[:, pl.ds(blk * UB, UB)], stg, sem)

            def drain(blk, stg, sem):
                pltpu.make_async_copy(
                    src_hbm.at[:, pl.ds(blk * UB, UB)], stg, sem).wait()

            def transpose(stg, tbuf):
                rots = [(iota + d) & (_LANES - 1) for d in range(_LANES)]

                def rows_body(rg, carry):
                    rbase = rg * (2 * _LANES)
                    for half in range(2):
                        for kk in range(D // _LANES):
                            fvec = iota + kk * _LANES
                            for d in range(_LANES):
                                rvec = rots[d] + (rbase + half * _LANES)
                                vals = plsc.load_gather(stg, [fvec, rvec])
                                sidx = rvec * D + fvec
                                plsc.store_scatter(tbuf, [sidx], vals)
                    return carry
                lax.fori_loop(0, UB // (2 * _LANES), rows_body, 0)

            @pl.when(wid < nfull)
            def _():
                fire(wid, stg_a, sem_a)

            def body(j, even, carry):
                blk = wid + j * NW
                nxt = blk + NW

                @pl.when(nxt < nfull)
                def _():
                    fire(nxt, stg_b if even else stg_a,
                         sem_b if even else sem_a)

                @pl.when(blk < nfull)
                def _():
                    cur_stg = stg_a if even else stg_b
                    cur_sem = sem_a if even else sem_b
                    cur_tbuf = tbuf_a if even else tbuf_b
                    cur_osem = sem_oa if even else sem_ob
                    drain(blk, cur_stg, cur_sem)

                    @pl.when(j >= 2)
                    def _():
                        pltpu.make_async_copy(
                            cur_tbuf,
                            dst_hbm.at[pl.ds(blk * UB * D, UB * D)],
                            cur_osem).wait()

                    transpose(cur_stg, cur_tbuf)
                    pltpu.async_copy(
                        cur_tbuf, dst_hbm.at[pl.ds(blk * UB * D, UB * D)],
                        cur_osem)
                return carry

            def body2(jj, carry):
                body(2 * jj, True, carry)
                body(2 * jj + 1, False, carry)
                return carry

            lax.fori_loop(0, (per_w + 1) // 2, body2, 0)

            @pl.when(wid < nfull)
            def _():
                pltpu.make_async_copy(
                    tbuf_a, dst_hbm.at[pl.ds(wid * UB * D, UB * D)],
                    sem_oa).wait()

            @pl.when(wid + NW < nfull)
            def _():
                pltpu.make_async_copy(
                    tbuf_b, dst_hbm.at[pl.ds(wid * UB * D, UB * D)],
                    sem_ob).wait()

            if rem:
                @pl.when(wid == nfull % NW)
                def _():
                    pltpu.sync_copy(tail_hbm, tbuf_a.at[pl.ds(0, rem * D)])
                    pltpu.sync_copy(
                        tbuf_a.at[pl.ds(0, rem * D)],
                        dst_hbm.at[pl.ds(nfull * UB * D, rem * D)])

        one_table(cvt_hbm, cv_tail, cv_out)

    return k1


def _build_main(B, H, D, V, NC, NS):
    NW = NC * NS
    BW = B // NW
    NCH = BW // _CHUNK_ROWS
    SW = _STREAM_ROWS * H
    CE = _CHUNK_ROWS * H

    mesh = plsc.VectorSubcoreMesh(core_axis_name="c", subcore_axis_name="s")

    group_offs = list(range(0, H - _LANES + 1, _LANES))
    if group_offs[-1] != H - _LANES:
        group_offs.append(H - _LANES)

    @functools.partial(
        pl.kernel,
        mesh=mesh,
        compiler_params=pltpu.CompilerParams(
            needs_layout_passes=False, use_tc_tiling_on_sc=False,
            disable_bounds_checks=True),
        out_type=jax.ShapeDtypeStruct((B, H), jnp.float32),
        scratch_types=[
            pltpu.VMEM((BW,), jnp.int32),
            pltpu.VMEM((BW // _STREAM_ROWS, SW), jnp.int32),
            pltpu.VMEM((BW, D), jnp.float32),
            pltpu.VMEM((CE, D), jnp.float32),
            pltpu.VMEM((CE, D), jnp.float32),
            pltpu.VMEM((BW, H), jnp.float32),
            pltpu.SemaphoreType.DMA,
            pltpu.SemaphoreType.DMA,
            pltpu.SemaphoreType.DMA,
        ],
    )
    def k2(uidx_hbm, purch_hbm, cv_hbm, u_hbm, out_hbm,
           uidx_v, purch_v, urows_v, buf_a, buf_b, out_v,
           sem_a, sem_b, sem_u):
        wid = lax.axis_index("s") * NC + lax.axis_index("c")
        base = wid * BW
        iota = jnp.arange(_LANES, dtype=jnp.int32)
        nstr = BW // _STREAM_ROWS
        pltpu.sync_copy(uidx_hbm.at[pl.ds(base, BW)], uidx_v)
        pltpu.sync_copy(purch_hbm.at[pl.ds(wid * nstr, nstr)], purch_v)

        def fire(j, buf, sem):
            for s in range(_CHUNK_STREAMS):
                pltpu.async_copy(
                    cv_hbm.at[purch_v.at[j * _CHUNK_STREAMS + s]],
                    buf.at[pl.ds(s * SW, SW)], sem)

        def drain(j, buf, sem):
            for s in range(_CHUNK_STREAMS):
                pltpu.make_async_copy(
                    cv_hbm.at[purch_v.at[j * _CHUNK_STREAMS + s]],
                    buf.at[pl.ds(s * SW, SW)], sem).wait()

        def compute(j, buf):
            def sub(s, carry):
                for r in range(_STREAM_ROWS):
                    row = j * _CHUNK_ROWS + s * _STREAM_ROWS + r
                    ce0 = s * SW + r * H
                    uvecs = [urows_v[row, pl.ds(kk * _LANES, _LANES)]
                             for kk in range(D // _LANES)]
                    for l0 in group_offs:
                        outvec = jnp.zeros((_LANES,), jnp.float32)
                        for i in range(_LANES):
                            e = ce0 + l0 + i
                            acc = buf[e, pl.ds(0, _LANES)] * uvecs[0]
                            for kk in range(1, D // _LANES):
                                acc = acc + buf[e, pl.ds(kk * _LANES, _LANES)] * uvecs[kk]
                            outvec = jnp.where(iota == i, jnp.sum(acc), outvec)
                        out_v[row, pl.ds(l0, _LANES)] = outvec
                return carry
            lax.fori_loop(0, _CHUNK_STREAMS, sub, 0)

        fire(0, buf_a, sem_a)
        pltpu.async_copy(u_hbm.at[uidx_v], urows_v, sem_u)
        pltpu.make_async_copy(u_hbm.at[uidx_v], urows_v, sem_u).wait()

        def body(jj, carry):
            j0 = 2 * jj
            j1 = 2 * jj + 1
            fire(j1, buf_b, sem_b)
            drain(j0, buf_a, sem_a)
            compute(j0, buf_a)

            @pl.when(j1 + 1 < NCH)
            def _():
                fire(j1 + 1, buf_a, sem_a)

            drain(j1, buf_b, sem_b)
            compute(j1, buf_b)
            return carry

        lax.fori_loop(0, NCH // 2, body, 0)
        pltpu.sync_copy(out_v, out_hbm.at[pl.ds(base, BW)])

    return k2


def kernel(user_idx, user_purchase, character_vector, u):
    B, H = user_purchase.shape
    V, D = character_vector.shape
    info = plsc.get_sparse_core_info()
    NC, NS = info.num_cores, info.num_subcores

    nfull = V // 128
    cv_tail = character_vector[nfull * 128:].reshape(-1)
    k1 = _build_transpose(V, D, NC, NS)
    cv_lin = k1(character_vector.T, cv_tail).reshape(V, D)

    purch2 = user_purchase.reshape(B // _STREAM_ROWS, _STREAM_ROWS * H)
    k2 = _build_main(B, H, D, V, NC, NS)
    return k2(user_idx, purch2, cv_lin, u)
